# baseline (speedup 1.0000x reference)
import functools

import jax
import jax.numpy as jnp
from jax import lax
from jax.experimental import pallas as pl
from jax.experimental.pallas import tpu as pltpu
from jax.experimental.pallas import tpu_sc as plsc

N_NODES = 10000
N_EDGES = 320000
D_NODE = 128
D_EDGE = 16
D_H = 128

NC = 2
NS = 16
NW = NC * NS
L = 16

CHUNK = 128
N_CHUNKS = N_EDGES // CHUNK
ROWS_PER_TILE = 624


def _prep_body(x_ref, w_ref, b_ref, o_ref):
    y = jnp.dot(x_ref[...], w_ref[...], preferred_element_type=jnp.float32)
    y = y + b_ref[...]
    col = lax.broadcasted_iota(jnp.int32, y.shape, 1)
    o_ref[...] = jnp.where(col < D_H, jnp.maximum(y, 0.0), y)


def _node_prep(nodes, W3, b3):
    grid = 10
    bm = N_NODES // grid
    return pl.pallas_call(
        _prep_body,
        grid=(grid,),
        in_specs=[
            pl.BlockSpec((bm, D_NODE), lambda i: (i, 0)),
            pl.BlockSpec((D_NODE, 3 * D_H), lambda i: (0, 0)),
            pl.BlockSpec((1, 3 * D_H), lambda i: (0, 0)),
        ],
        out_specs=pl.BlockSpec((bm, 3 * D_H), lambda i: (i, 0)),
        out_shape=jax.ShapeDtypeStruct((N_NODES, 3 * D_H), jnp.float32),
    )(nodes, W3, b3)


def _mm_bias_body(x_ref, w_ref, b_ref, o_ref):
    o_ref[...] = (
        jnp.dot(x_ref[...], w_ref[...], preferred_element_type=jnp.float32)
        + b_ref[...]
    )


def _edge_matmul(x, w, b, grid=32):
    e, k = x.shape
    n = w.shape[1]
    bm = e // grid
    return pl.pallas_call(
        _mm_bias_body,
        grid=(grid,),
        in_specs=[
            pl.BlockSpec((bm, k), lambda i: (i, 0)),
            pl.BlockSpec((k, n), lambda i: (0, 0)),
            pl.BlockSpec((1, n), lambda i: (0, 0)),
        ],
        out_specs=pl.BlockSpec((bm, n), lambda i: (i, 0)),
        out_shape=jax.ShapeDtypeStruct((e, n), jnp.float32),
    )(x, w, b)


def _node_update_body(n_ref, ss_ref, sr_ref, cs_ref, cr_ref, wv_ref, bv_ref,
                      wus_ref, wur_ref, n_out, qs_out, qr_out):
    cs = jnp.maximum(cs_ref[:, :1], 1.0)
    cr = jnp.maximum(cr_ref[:, :1], 1.0)
    x = jnp.concatenate([n_ref[...], ss_ref[...] / cs, sr_ref[...] / cr],
                        axis=1)
    h = jnp.maximum(
        jnp.dot(x, wv_ref[...], preferred_element_type=jnp.float32)
        + bv_ref[...], 0.0)
    n_out[...] = h
    qs_out[...] = jnp.dot(h, wus_ref[...], preferred_element_type=jnp.float32)
    qr_out[...] = jnp.dot(h, wur_ref[...], preferred_element_type=jnp.float32)


def _node_update(n, ss, sr, cs, cr, Wv, bv, Wus, Wur):
    grid = 10
    bm = N_NODES // grid
    return pl.pallas_call(
        _node_update_body,
        grid=(grid,),
        in_specs=[
            pl.BlockSpec((bm, D_H), lambda i: (i, 0)),
            pl.BlockSpec((bm, D_H), lambda i: (i, 0)),
            pl.BlockSpec((bm, D_H), lambda i: (i, 0)),
            pl.BlockSpec((bm, D_H), lambda i: (i, 0)),
            pl.BlockSpec((bm, D_H), lambda i: (i, 0)),
            pl.BlockSpec((3 * D_H, D_H), lambda i: (0, 0)),
            pl.BlockSpec((1, D_H), lambda i: (0, 0)),
            pl.BlockSpec((D_H, D_H), lambda i: (0, 0)),
            pl.BlockSpec((D_H, D_H), lambda i: (0, 0)),
        ],
        out_specs=[
            pl.BlockSpec((bm, D_H), lambda i: (i, 0)),
            pl.BlockSpec((bm, D_H), lambda i: (i, 0)),
            pl.BlockSpec((bm, D_H), lambda i: (i, 0)),
        ],
        out_shape=[
            jax.ShapeDtypeStruct((N_NODES, D_H), jnp.float32),
            jax.ShapeDtypeStruct((N_NODES, D_H), jnp.float32),
            jax.ShapeDtypeStruct((N_NODES, D_H), jnp.float32),
        ],
    )(n, ss, sr, cs, cr, Wv, bv, Wus, Wur)


def _sc_mesh():
    return plsc.VectorSubcoreMesh(core_axis_name="c", subcore_axis_name="s",
                                  num_cores=NC, num_subcores=NS)


CPW = (N_CHUNKS + NW - 1) // NW
CPW_LAST = N_CHUNKS - (NW - 1) * CPW


def _gather_pipeline(emm, qs, qr, senders, receivers,
                     idx_s_all, idx_r_all, acc, rows_s, rows_r,
                     sem_e, sem_s, sem_r, process,
                     chunk=CHUNK, n_chunks=N_CHUNKS):
    wid = lax.axis_index("s") * NC + lax.axis_index("c")
    cpw = (n_chunks + NW - 1) // NW
    cpw_last = n_chunks - (NW - 1) * cpw
    first = wid * cpw

    @pl.when(wid < NW - 1)
    def _():
        pltpu.sync_copy(senders.at[pl.ds(first * chunk, cpw * chunk)],
                        idx_s_all)
        pltpu.sync_copy(receivers.at[pl.ds(first * chunk, cpw * chunk)],
                        idx_r_all)

    @pl.when(wid == NW - 1)
    def _():
        pltpu.sync_copy(senders.at[pl.ds(first * chunk, cpw_last * chunk)],
                        idx_s_all.at[pl.ds(0, cpw_last * chunk)])
        pltpu.sync_copy(receivers.at[pl.ds(first * chunk, cpw_last * chunk)],
                        idx_r_all.at[pl.ds(0, cpw_last * chunk)])

    def valid(k):
        return (k < cpw) & (first + k < n_chunks)

    def issue(k, b):
        @pl.when(valid(k))
        def _():
            base = (first + k) * chunk
            pltpu.async_copy(emm.at[pl.ds(base, chunk)], acc[b], sem_e[b])
            pltpu.async_copy(qs.at[idx_s_all.at[pl.ds(k * chunk, chunk)]],
                             rows_s[b], sem_s[b])
            pltpu.async_copy(qr.at[idx_r_all.at[pl.ds(k * chunk, chunk)]],
                             rows_r[b], sem_r[b])

    def finish(k, b):
        @pl.when(valid(k))
        def _():
            base = (first + k) * chunk
            pltpu.make_async_copy(emm.at[pl.ds(base, chunk)], acc[b],
                                  sem_e[b]).wait()
            pltpu.make_async_copy(
                qs.at[idx_s_all.at[pl.ds(k * chunk, chunk)]], rows_s[b],
                sem_s[b]).wait()
            pltpu.make_async_copy(
                qr.at[idx_r_all.at[pl.ds(k * chunk, chunk)]], rows_r[b],
                sem_r[b]).wait()
            process(b, base)

    issue(0, 0)

    def pair_body(m, _):
        k = 2 * m
        issue(k + 1, 1)
        finish(k, 0)
        issue(k + 2, 0)
        finish(k + 1, 1)
        return 0

    lax.fori_loop(0, (cpw + 1) // 2, pair_body, 0)


def _assemble_kernel(emm, qs, qr, senders, receivers, out,
                     idx_s_all, idx_r_all, acc, rows_s, rows_r,
                     sem_e, sem_s, sem_r):

    def process(b, base):
        def row_body(i, _):
            for j in range(D_H // L):
                sl = pl.ds(j * L, L)
                acc[b][i, sl] = jnp.maximum(
                    acc[b][i, sl] + rows_s[b][i, sl] + rows_r[b][i, sl],
                    0.0)
            return 0

        lax.fori_loop(0, CHUNK, row_body, 0)
        pltpu.sync_copy(acc[b], out.at[pl.ds(base, CHUNK)])

    _gather_pipeline(emm, qs, qr, senders, receivers,
                     idx_s_all, idx_r_all, acc, rows_s, rows_r,
                     sem_e, sem_s, sem_r, process)


_ASM_SCRATCH = [
    pltpu.VMEM((CPW * CHUNK,), jnp.int32),
    pltpu.VMEM((CPW * CHUNK,), jnp.int32),
    [pltpu.VMEM((CHUNK, D_H), jnp.float32) for _ in range(2)],
    [pltpu.VMEM((CHUNK, D_H), jnp.float32) for _ in range(2)],
    [pltpu.VMEM((CHUNK, D_H), jnp.float32) for _ in range(2)],
    [pltpu.SemaphoreType.DMA for _ in range(2)],
    [pltpu.SemaphoreType.DMA for _ in range(2)],
    [pltpu.SemaphoreType.DMA for _ in range(2)],
]


def _sc_assemble(emm, qs, qr, senders, receivers):
    return pl.kernel(
        _assemble_kernel,
        out_type=jax.ShapeDtypeStruct((N_EDGES, D_H), jnp.float32),
        mesh=_sc_mesh(),
        scratch_types=_ASM_SCRATCH,
    )(emm, qs, qr, senders, receivers)


CHUNK_F = 64
N_CHUNKS_F = N_EDGES // CHUNK_F
CPW_F = (N_CHUNKS_F + NW - 1) // NW


def _assemble_final_kernel(emm, qs, qr, senders, receivers, wf, out,
                           idx_s_all, idx_r_all, acc, rows_s, rows_r,
                           sem_e, sem_s, sem_r, wbuf, psum):
    pltpu.sync_copy(wf, wbuf)

    def process(b, base):
        def row_body(i, _):
            dsum = jnp.zeros((L,), jnp.float32)
            for j in range(D_H // L):
                sl = pl.ds(j * L, L)
                v = jnp.maximum(
                    acc[b][i, sl] + rows_s[b][i, sl] + rows_r[b][i, sl],
                    0.0)
                dsum = dsum + v * wbuf[sl]
            psum[b][i, pl.ds(0, L)] = dsum
            return 0

        lax.fori_loop(0, CHUNK_F, row_body, 0)
        pltpu.sync_copy(psum[b], out.at[pl.ds(base, CHUNK_F)])

    _gather_pipeline(emm, qs, qr, senders, receivers,
                     idx_s_all, idx_r_all, acc, rows_s, rows_r,
                     sem_e, sem_s, sem_r, process,
                     chunk=CHUNK_F, n_chunks=N_CHUNKS_F)


def _sc_assemble_final(emm, qs, qr, senders, receivers, wf):
    return pl.kernel(
        _assemble_final_kernel,
        out_type=jax.ShapeDtypeStruct((N_EDGES, L), jnp.float32),
        mesh=_sc_mesh(),
        scratch_types=[
            pltpu.VMEM((CPW_F * CHUNK_F,), jnp.int32),
            pltpu.VMEM((CPW_F * CHUNK_F,), jnp.int32),
            [pltpu.VMEM((CHUNK_F, D_H), jnp.float32) for _ in range(2)],
            [pltpu.VMEM((CHUNK_F, D_H), jnp.float32) for _ in range(2)],
            [pltpu.VMEM((CHUNK_F, D_H), jnp.float32) for _ in range(2)],
            [pltpu.SemaphoreType.DMA for _ in range(2)],
            [pltpu.SemaphoreType.DMA for _ in range(2)],
            [pltpu.SemaphoreType.DMA for _ in range(2)],
            pltpu.VMEM((D_H,), jnp.float32),
            [pltpu.VMEM((CHUNK_F, L), jnp.float32) for _ in range(2)],
        ],
    )(emm, qs, qr, senders, receivers, wf)


def _tile_rows(s, fn):
    for off in (0, 128, 256, 384):
        fn(off, 128)

    @pl.when(s < NS - 1)
    def _():
        fn(512, 112)

    @pl.when(s == NS - 1)
    def _():
        fn(512, 128)


CPT = (N_CHUNKS + NS - 1) // NS


def _scatter_kernel(e, idx_both, s_out, acc_sh, ebuf, idx2, sem_e, sem_i):
    c = lax.axis_index("c")
    s = lax.axis_index("s")

    def zrow(i, _):
        for j in range(D_H // L):
            ebuf[0][i, pl.ds(j * L, L)] = jnp.zeros((L,), jnp.float32)
        return 0

    lax.fori_loop(0, CHUNK, zrow, 0)
    base = s * ROWS_PER_TILE

    def zcopy(off, nrows):
        pltpu.sync_copy(ebuf[0].at[pl.ds(0, nrows)],
                        acc_sh.at[pl.ds(base + off, nrows)])

    _tile_rows(s, zcopy)
    plsc.subcore_barrier()

    first = s * CPT

    def valid(k):
        return (k < CPT) & (first + k < N_CHUNKS)

    def issue(k, b):
        @pl.when(valid(k))
        def _():
            ebase = (first + k) * CHUNK
            pltpu.async_copy(e.at[pl.ds(ebase, CHUNK)], ebuf[b], sem_e[b])
            pltpu.async_copy(idx_both.at[c, pl.ds(ebase, CHUNK)],
                             idx2[b].at[0], sem_i[b])

    def finish(k, b):
        @pl.when(valid(k))
        def _():
            ebase = (first + k) * CHUNK
            pltpu.make_async_copy(e.at[pl.ds(ebase, CHUNK)], ebuf[b],
                                  sem_e[b]).wait()
            pltpu.make_async_copy(idx_both.at[c, pl.ds(ebase, CHUNK)],
                                  idx2[b].at[0], sem_i[b]).wait()
            pltpu.sync_copy(ebuf[b], acc_sh.at[idx2[b].at[0]], add=True)

    issue(0, 0)

    def pair_body(m, _):
        k = 2 * m
        issue(k + 1, 1)
        finish(k, 0)
        issue(k + 2, 0)
        finish(k + 1, 1)
        return 0

    lax.fori_loop(0, (CPT + 1) // 2, pair_body, 0)
    plsc.subcore_barrier()

    def wcopy(off, nrows):
        pltpu.sync_copy(acc_sh.at[pl.ds(base + off, nrows)],
                        ebuf[0].at[pl.ds(0, nrows)])
        pltpu.sync_copy(ebuf[0].at[pl.ds(0, nrows)],
                        s_out.at[c, pl.ds(base + off, nrows)])

    _tile_rows(s, wcopy)


def _sc_scatter(e, idx_both):
    return pl.kernel(
        _scatter_kernel,
        out_type=jax.ShapeDtypeStruct((NC, N_NODES, D_H), jnp.float32),
        mesh=_sc_mesh(),
        scratch_types=[
            pltpu.VMEM_SHARED((N_NODES, D_H), jnp.float32),
            [pltpu.VMEM((CHUNK, D_H), jnp.float32) for _ in range(2)],
            [pltpu.VMEM((1, CHUNK), jnp.int32) for _ in range(2)],
            [pltpu.SemaphoreType.DMA for _ in range(2)],
            [pltpu.SemaphoreType.DMA for _ in range(2)],
        ],
    )(e, idx_both)


def _count_kernel(idx_both, c_out, cnt_sh, cbuf, idx2, sem_i):
    c = lax.axis_index("c")
    s = lax.axis_index("s")

    def zrow(i, _):
        for j in range(D_H // L):
            cbuf[i, pl.ds(j * L, L)] = jnp.zeros((L,), jnp.float32)
        return 0

    lax.fori_loop(0, CHUNK, zrow, 0)
    base = s * ROWS_PER_TILE

    def zcopy(off, nrows):
        pltpu.sync_copy(cbuf.at[pl.ds(0, nrows)],
                        cnt_sh.at[pl.ds(base + off, nrows)])

    _tile_rows(s, zcopy)
    plsc.subcore_barrier()

    def orow(i, _):
        for j in range(D_H // L):
            cbuf[i, pl.ds(j * L, L)] = jnp.ones((L,), jnp.float32)
        return 0

    lax.fori_loop(0, CHUNK, orow, 0)

    first = s * CPT

    def valid(k):
        return (k < CPT) & (first + k < N_CHUNKS)

    def issue(k, b):
        @pl.when(valid(k))
        def _():
            ebase = (first + k) * CHUNK
            pltpu.async_copy(idx_both.at[c, pl.ds(ebase, CHUNK)],
                             idx2[b].at[0], sem_i[b])

    def finish(k, b):
        @pl.when(valid(k))
        def _():
            ebase = (first + k) * CHUNK
            pltpu.make_async_copy(idx_both.at[c, pl.ds(ebase, CHUNK)],
                                  idx2[b].at[0], sem_i[b]).wait()
            pltpu.sync_copy(cbuf, cnt_sh.at[idx2[b].at[0]], add=True)

    issue(0, 0)

    def pair_body(m, _):
        k = 2 * m
        issue(k + 1, 1)
        finish(k, 0)
        issue(k + 2, 0)
        finish(k + 1, 1)
        return 0

    lax.fori_loop(0, (CPT + 1) // 2, pair_body, 0)
    plsc.subcore_barrier()

    def wcopy(off, nrows):
        pltpu.sync_copy(cnt_sh.at[pl.ds(base + off, nrows)],
                        cbuf.at[pl.ds(0, nrows)])
        pltpu.sync_copy(cbuf.at[pl.ds(0, nrows)],
                        c_out.at[c, pl.ds(base + off, nrows)])

    _tile_rows(s, wcopy)


def _sc_count(idx_both):
    return pl.kernel(
        _count_kernel,
        out_type=jax.ShapeDtypeStruct((NC, N_NODES, D_H), jnp.float32),
        mesh=_sc_mesh(),
        scratch_types=[
            pltpu.VMEM_SHARED((N_NODES, D_H), jnp.float32),
            pltpu.VMEM((CHUNK, D_H), jnp.float32),
            [pltpu.VMEM((1, CHUNK), jnp.int32) for _ in range(2)],
            [pltpu.SemaphoreType.DMA for _ in range(2)],
        ],
    )(idx_both)


def kernel(nodes, edges, senders, receivers, Wn, bn, We, be, Wv, bv, Wu, bu,
           Wf, bf):
    senders = senders.astype(jnp.int32)
    receivers = receivers.astype(jnp.int32)

    We_e = We[:D_EDGE]
    We_s = We[D_EDGE:D_EDGE + D_NODE]
    We_r = We[D_EDGE + D_NODE:]
    Wu_e, Wu_s, Wu_r = Wu[:D_H], Wu[D_H:2 * D_H], Wu[2 * D_H:]

    W3 = jnp.concatenate([Wn, We_s, We_r], axis=1)
    b3 = jnp.concatenate([bn, jnp.zeros((2 * D_H,), jnp.float32)])[None, :]

    prep = _node_prep(nodes, W3, b3)
    n = prep[:, :D_H]
    ps, pr = prep[:, D_H:2 * D_H], prep[:, 2 * D_H:]

    idx_both = jnp.stack([senders, receivers])
    c_acc = _sc_count(idx_both)

    e_base = _edge_matmul(edges, We_e, be[None, :])
    e = _sc_assemble(e_base, ps, pr, senders, receivers)

    for p in range(2):
        s_acc = _sc_scatter(e, idx_both)
        emm = _edge_matmul(e, Wu_e, bu[None, :])
        n, qs, qr = _node_update(n, s_acc[0], s_acc[1], c_acc[0], c_acc[1],
                                 Wv, bv[None, :], Wu_s, Wu_r)
        if p == 0:
            e = _sc_assemble(emm, qs, qr, senders, receivers)
        else:
            psum = _sc_assemble_final(emm, qs, qr, senders, receivers,
                                      Wf[:, 0])

    vij = _edge_matmul(psum, jnp.ones((L, 1), jnp.float32), bf[None, :])
    return vij[:, 0]


# speedup vs baseline: 1.0218x; 1.0218x over previous
import functools

import jax
import jax.numpy as jnp
from jax import lax
from jax.experimental import pallas as pl
from jax.experimental.pallas import tpu as pltpu
from jax.experimental.pallas import tpu_sc as plsc

N_NODES = 10000
N_EDGES = 320000
D_NODE = 128
D_EDGE = 16
D_H = 128

NC = 2
NS = 16
NW = NC * NS
L = 16

CHUNK = 128
N_CHUNKS = N_EDGES // CHUNK
ROWS_PER_TILE = 624


def _prep_body(x_ref, w_ref, b_ref, o_ref):
    y = jnp.dot(x_ref[...], w_ref[...], preferred_element_type=jnp.float32)
    y = y + b_ref[...]
    col = lax.broadcasted_iota(jnp.int32, y.shape, 1)
    o_ref[...] = jnp.where(col < D_H, jnp.maximum(y, 0.0), y)


def _node_prep(nodes, W3, b3):
    grid = 10
    bm = N_NODES // grid
    return pl.pallas_call(
        _prep_body,
        grid=(grid,),
        in_specs=[
            pl.BlockSpec((bm, D_NODE), lambda i: (i, 0)),
            pl.BlockSpec((D_NODE, 3 * D_H), lambda i: (0, 0)),
            pl.BlockSpec((1, 3 * D_H), lambda i: (0, 0)),
        ],
        out_specs=pl.BlockSpec((bm, 3 * D_H), lambda i: (i, 0)),
        out_shape=jax.ShapeDtypeStruct((N_NODES, 3 * D_H), jnp.float32),
    )(nodes, W3, b3)


def _mm_bias_body(x_ref, w_ref, b_ref, o_ref):
    o_ref[...] = (
        jnp.dot(x_ref[...], w_ref[...], preferred_element_type=jnp.float32)
        + b_ref[...]
    )


def _edge_matmul(x, w, b, grid=32):
    e, k = x.shape
    n = w.shape[1]
    bm = e // grid
    return pl.pallas_call(
        _mm_bias_body,
        grid=(grid,),
        in_specs=[
            pl.BlockSpec((bm, k), lambda i: (i, 0)),
            pl.BlockSpec((k, n), lambda i: (0, 0)),
            pl.BlockSpec((1, n), lambda i: (0, 0)),
        ],
        out_specs=pl.BlockSpec((bm, n), lambda i: (i, 0)),
        out_shape=jax.ShapeDtypeStruct((e, n), jnp.float32),
    )(x, w, b)


def _node_update_body(n_ref, ss_ref, sr_ref, cs_ref, cr_ref, wv_ref, bv_ref,
                      wus_ref, wur_ref, n_out, qs_out, qr_out):
    cs = jnp.maximum(cs_ref[:, :1], 1.0)
    cr = jnp.maximum(cr_ref[:, :1], 1.0)
    x = jnp.concatenate([n_ref[...], ss_ref[...] / cs, sr_ref[...] / cr],
                        axis=1)
    h = jnp.maximum(
        jnp.dot(x, wv_ref[...], preferred_element_type=jnp.float32)
        + bv_ref[...], 0.0)
    n_out[...] = h
    qs_out[...] = jnp.dot(h, wus_ref[...], preferred_element_type=jnp.float32)
    qr_out[...] = jnp.dot(h, wur_ref[...], preferred_element_type=jnp.float32)


def _node_update(n, ss, sr, cs, cr, Wv, bv, Wus, Wur):
    grid = 10
    bm = N_NODES // grid
    return pl.pallas_call(
        _node_update_body,
        grid=(grid,),
        in_specs=[
            pl.BlockSpec((bm, D_H), lambda i: (i, 0)),
            pl.BlockSpec((bm, D_H), lambda i: (i, 0)),
            pl.BlockSpec((bm, D_H), lambda i: (i, 0)),
            pl.BlockSpec((bm, D_H), lambda i: (i, 0)),
            pl.BlockSpec((bm, D_H), lambda i: (i, 0)),
            pl.BlockSpec((3 * D_H, D_H), lambda i: (0, 0)),
            pl.BlockSpec((1, D_H), lambda i: (0, 0)),
            pl.BlockSpec((D_H, D_H), lambda i: (0, 0)),
            pl.BlockSpec((D_H, D_H), lambda i: (0, 0)),
        ],
        out_specs=[
            pl.BlockSpec((bm, D_H), lambda i: (i, 0)),
            pl.BlockSpec((bm, D_H), lambda i: (i, 0)),
            pl.BlockSpec((bm, D_H), lambda i: (i, 0)),
        ],
        out_shape=[
            jax.ShapeDtypeStruct((N_NODES, D_H), jnp.float32),
            jax.ShapeDtypeStruct((N_NODES, D_H), jnp.float32),
            jax.ShapeDtypeStruct((N_NODES, D_H), jnp.float32),
        ],
    )(n, ss, sr, cs, cr, Wv, bv, Wus, Wur)


def _sc_mesh():
    return plsc.VectorSubcoreMesh(core_axis_name="c", subcore_axis_name="s",
                                  num_cores=NC, num_subcores=NS)


CPW = (N_CHUNKS + NW - 1) // NW
CPW_LAST = N_CHUNKS - (NW - 1) * CPW


def _gather_pipeline(emm, qs, qr, senders, receivers,
                     idx_s_all, idx_r_all, acc, rows_s, rows_r,
                     sem_e, sem_s, sem_r, process,
                     chunk=CHUNK, n_chunks=N_CHUNKS):
    wid = lax.axis_index("s") * NC + lax.axis_index("c")
    cpw = (n_chunks + NW - 1) // NW
    cpw_last = n_chunks - (NW - 1) * cpw
    first = wid * cpw

    @pl.when(wid < NW - 1)
    def _():
        pltpu.sync_copy(senders.at[pl.ds(first * chunk, cpw * chunk)],
                        idx_s_all)
        pltpu.sync_copy(receivers.at[pl.ds(first * chunk, cpw * chunk)],
                        idx_r_all)

    @pl.when(wid == NW - 1)
    def _():
        pltpu.sync_copy(senders.at[pl.ds(first * chunk, cpw_last * chunk)],
                        idx_s_all.at[pl.ds(0, cpw_last * chunk)])
        pltpu.sync_copy(receivers.at[pl.ds(first * chunk, cpw_last * chunk)],
                        idx_r_all.at[pl.ds(0, cpw_last * chunk)])

    def valid(k):
        return (k < cpw) & (first + k < n_chunks)

    def issue(k, b):
        @pl.when(valid(k))
        def _():
            base = (first + k) * chunk
            pltpu.async_copy(emm.at[pl.ds(base, chunk)], acc[b], sem_e[b])
            pltpu.async_copy(qs.at[idx_s_all.at[pl.ds(k * chunk, chunk)]],
                             rows_s[b], sem_s[b])
            pltpu.async_copy(qr.at[idx_r_all.at[pl.ds(k * chunk, chunk)]],
                             rows_r[b], sem_r[b])

    def finish(k, b):
        @pl.when(valid(k))
        def _():
            base = (first + k) * chunk
            pltpu.make_async_copy(emm.at[pl.ds(base, chunk)], acc[b],
                                  sem_e[b]).wait()
            pltpu.make_async_copy(
                qs.at[idx_s_all.at[pl.ds(k * chunk, chunk)]], rows_s[b],
                sem_s[b]).wait()
            pltpu.make_async_copy(
                qr.at[idx_r_all.at[pl.ds(k * chunk, chunk)]], rows_r[b],
                sem_r[b]).wait()
            process(b, base)

    issue(0, 0)

    def pair_body(m, _):
        k = 2 * m
        issue(k + 1, 1)
        finish(k, 0)
        issue(k + 2, 0)
        finish(k + 1, 1)
        return 0

    lax.fori_loop(0, (cpw + 1) // 2, pair_body, 0)


def _assemble_kernel(emm, qs, qr, senders, receivers, out,
                     idx_s_all, idx_r_all, acc, rows_s, rows_r,
                     sem_e, sem_s, sem_r):

    def process(b, base):
        def row_body(i, _):
            for j in range(D_H // L):
                sl = pl.ds(j * L, L)
                acc[b][i, sl] = jnp.maximum(
                    acc[b][i, sl] + rows_s[b][i, sl] + rows_r[b][i, sl],
                    0.0)
            return 0

        lax.fori_loop(0, CHUNK, row_body, 0)
        pltpu.sync_copy(acc[b], out.at[pl.ds(base, CHUNK)])

    _gather_pipeline(emm, qs, qr, senders, receivers,
                     idx_s_all, idx_r_all, acc, rows_s, rows_r,
                     sem_e, sem_s, sem_r, process)


_ASM_SCRATCH = [
    pltpu.VMEM((CPW * CHUNK,), jnp.int32),
    pltpu.VMEM((CPW * CHUNK,), jnp.int32),
    [pltpu.VMEM((CHUNK, D_H), jnp.float32) for _ in range(2)],
    [pltpu.VMEM((CHUNK, D_H), jnp.float32) for _ in range(2)],
    [pltpu.VMEM((CHUNK, D_H), jnp.float32) for _ in range(2)],
    [pltpu.SemaphoreType.DMA for _ in range(2)],
    [pltpu.SemaphoreType.DMA for _ in range(2)],
    [pltpu.SemaphoreType.DMA for _ in range(2)],
]


def _sc_assemble(emm, qs, qr, senders, receivers):
    return pl.kernel(
        _assemble_kernel,
        out_type=jax.ShapeDtypeStruct((N_EDGES, D_H), jnp.float32),
        mesh=_sc_mesh(),
        scratch_types=_ASM_SCRATCH,
    )(emm, qs, qr, senders, receivers)


def _tile_rows(s, fn):
    for off in (0, 128, 256, 384):
        fn(off, 128)

    @pl.when(s < NS - 1)
    def _():
        fn(512, 112)

    @pl.when(s == NS - 1)
    def _():
        fn(512, 128)


CPT = (N_CHUNKS + NS - 1) // NS


def _scatter_kernel(e, idx_both, s_out, acc_sh, ebuf, idx2, sem_e, sem_i):
    c = lax.axis_index("c")
    s = lax.axis_index("s")

    def zrow(i, _):
        for j in range(D_H // L):
            ebuf[0][i, pl.ds(j * L, L)] = jnp.zeros((L,), jnp.float32)
        return 0

    lax.fori_loop(0, CHUNK, zrow, 0)
    base = s * ROWS_PER_TILE

    def zcopy(off, nrows):
        pltpu.sync_copy(ebuf[0].at[pl.ds(0, nrows)],
                        acc_sh.at[pl.ds(base + off, nrows)])

    _tile_rows(s, zcopy)
    plsc.subcore_barrier()

    first = s * CPT

    def valid(k):
        return (k < CPT) & (first + k < N_CHUNKS)

    def issue(k, b):
        @pl.when(valid(k))
        def _():
            ebase = (first + k) * CHUNK
            pltpu.async_copy(e.at[pl.ds(ebase, CHUNK)], ebuf[b], sem_e[b])
            pltpu.async_copy(idx_both.at[c, pl.ds(ebase, CHUNK)],
                             idx2[b].at[0], sem_i[b])

    def finish(k, b):
        @pl.when(valid(k))
        def _():
            ebase = (first + k) * CHUNK
            pltpu.make_async_copy(e.at[pl.ds(ebase, CHUNK)], ebuf[b],
                                  sem_e[b]).wait()
            pltpu.make_async_copy(idx_both.at[c, pl.ds(ebase, CHUNK)],
                                  idx2[b].at[0], sem_i[b]).wait()
            pltpu.sync_copy(ebuf[b], acc_sh.at[idx2[b].at[0]], add=True)

    issue(0, 0)

    def pair_body(m, _):
        k = 2 * m
        issue(k + 1, 1)
        finish(k, 0)
        issue(k + 2, 0)
        finish(k + 1, 1)
        return 0

    lax.fori_loop(0, (CPT + 1) // 2, pair_body, 0)
    plsc.subcore_barrier()

    def wcopy(off, nrows):
        pltpu.sync_copy(acc_sh.at[pl.ds(base + off, nrows)],
                        ebuf[0].at[pl.ds(0, nrows)])
        pltpu.sync_copy(ebuf[0].at[pl.ds(0, nrows)],
                        s_out.at[c, pl.ds(base + off, nrows)])

    _tile_rows(s, wcopy)


def _sc_scatter(e, idx_both):
    return pl.kernel(
        _scatter_kernel,
        out_type=jax.ShapeDtypeStruct((NC, N_NODES, D_H), jnp.float32),
        mesh=_sc_mesh(),
        scratch_types=[
            pltpu.VMEM_SHARED((N_NODES, D_H), jnp.float32),
            [pltpu.VMEM((CHUNK, D_H), jnp.float32) for _ in range(2)],
            [pltpu.VMEM((1, CHUNK), jnp.int32) for _ in range(2)],
            [pltpu.SemaphoreType.DMA for _ in range(2)],
            [pltpu.SemaphoreType.DMA for _ in range(2)],
        ],
    )(e, idx_both)


def _count_kernel(idx_both, c_out, cnt_sh, cbuf, idx2, sem_i):
    c = lax.axis_index("c")
    s = lax.axis_index("s")

    def zrow(i, _):
        for j in range(D_H // L):
            cbuf[i, pl.ds(j * L, L)] = jnp.zeros((L,), jnp.float32)
        return 0

    lax.fori_loop(0, CHUNK, zrow, 0)
    base = s * ROWS_PER_TILE

    def zcopy(off, nrows):
        pltpu.sync_copy(cbuf.at[pl.ds(0, nrows)],
                        cnt_sh.at[pl.ds(base + off, nrows)])

    _tile_rows(s, zcopy)
    plsc.subcore_barrier()

    def orow(i, _):
        for j in range(D_H // L):
            cbuf[i, pl.ds(j * L, L)] = jnp.ones((L,), jnp.float32)
        return 0

    lax.fori_loop(0, CHUNK, orow, 0)

    first = s * CPT

    def valid(k):
        return (k < CPT) & (first + k < N_CHUNKS)

    def issue(k, b):
        @pl.when(valid(k))
        def _():
            ebase = (first + k) * CHUNK
            pltpu.async_copy(idx_both.at[c, pl.ds(ebase, CHUNK)],
                             idx2[b].at[0], sem_i[b])

    def finish(k, b):
        @pl.when(valid(k))
        def _():
            ebase = (first + k) * CHUNK
            pltpu.make_async_copy(idx_both.at[c, pl.ds(ebase, CHUNK)],
                                  idx2[b].at[0], sem_i[b]).wait()
            pltpu.sync_copy(cbuf, cnt_sh.at[idx2[b].at[0]], add=True)

    issue(0, 0)

    def pair_body(m, _):
        k = 2 * m
        issue(k + 1, 1)
        finish(k, 0)
        issue(k + 2, 0)
        finish(k + 1, 1)
        return 0

    lax.fori_loop(0, (CPT + 1) // 2, pair_body, 0)
    plsc.subcore_barrier()

    def wcopy(off, nrows):
        pltpu.sync_copy(cnt_sh.at[pl.ds(base + off, nrows)],
                        cbuf.at[pl.ds(0, nrows)])
        pltpu.sync_copy(cbuf.at[pl.ds(0, nrows)],
                        c_out.at[c, pl.ds(base + off, nrows)])

    _tile_rows(s, wcopy)


def _sc_count(idx_both):
    return pl.kernel(
        _count_kernel,
        out_type=jax.ShapeDtypeStruct((NC, N_NODES, D_H), jnp.float32),
        mesh=_sc_mesh(),
        scratch_types=[
            pltpu.VMEM_SHARED((N_NODES, D_H), jnp.float32),
            pltpu.VMEM((CHUNK, D_H), jnp.float32),
            [pltpu.VMEM((1, CHUNK), jnp.int32) for _ in range(2)],
            [pltpu.SemaphoreType.DMA for _ in range(2)],
        ],
    )(idx_both)


def kernel(nodes, edges, senders, receivers, Wn, bn, We, be, Wv, bv, Wu, bu,
           Wf, bf):
    senders = senders.astype(jnp.int32)
    receivers = receivers.astype(jnp.int32)

    We_e = We[:D_EDGE]
    We_s = We[D_EDGE:D_EDGE + D_NODE]
    We_r = We[D_EDGE + D_NODE:]
    Wu_e, Wu_s, Wu_r = Wu[:D_H], Wu[D_H:2 * D_H], Wu[2 * D_H:]

    W3 = jnp.concatenate([Wn, We_s, We_r], axis=1)
    b3 = jnp.concatenate([bn, jnp.zeros((2 * D_H,), jnp.float32)])[None, :]

    prep = _node_prep(nodes, W3, b3)
    n = prep[:, :D_H]
    ps, pr = prep[:, D_H:2 * D_H], prep[:, 2 * D_H:]

    idx_both = jnp.stack([senders, receivers])
    c_acc = _sc_count(idx_both)

    e_base = _edge_matmul(edges, We_e, be[None, :])
    e = _sc_assemble(e_base, ps, pr, senders, receivers)

    for _ in range(2):
        s_acc = _sc_scatter(e, idx_both)
        emm = _edge_matmul(e, Wu_e, bu[None, :])
        n, qs, qr = _node_update(n, s_acc[0], s_acc[1], c_acc[0], c_acc[1],
                                 Wv, bv[None, :], Wu_s, Wu_r)
        e = _sc_assemble(emm, qs, qr, senders, receivers)

    vij = _edge_matmul(e, Wf, bf[None, :])
    return vij[:, 0]


# speedup vs baseline: 1.0221x; 1.0003x over previous
import jax
import jax.numpy as jnp
from jax import lax
from jax.experimental import pallas as pl
from jax.experimental.pallas import tpu as pltpu
from jax.experimental.pallas import tpu_sc as plsc

N_NODES = 10000
N_EDGES = 320000
D_NODE = 128
D_EDGE = 16
D_H = 128

NC = 2
NS = 16
NW = NC * NS
L = 16

CHUNK = 128
N_CHUNKS = N_EDGES // CHUNK
ROWS_PER_TILE = 624


def _prep_body(x_ref, w_ref, b_ref, o_ref):
    y = jnp.dot(x_ref[...], w_ref[...], preferred_element_type=jnp.float32)
    y = y + b_ref[...]
    col = lax.broadcasted_iota(jnp.int32, y.shape, 1)
    o_ref[...] = jnp.where(col < D_H, jnp.maximum(y, 0.0), y)


def _node_prep(nodes, W3, b3):
    grid = 10
    bm = N_NODES // grid
    return pl.pallas_call(
        _prep_body,
        grid=(grid,),
        in_specs=[
            pl.BlockSpec((bm, D_NODE), lambda i: (i, 0)),
            pl.BlockSpec((D_NODE, 3 * D_H), lambda i: (0, 0)),
            pl.BlockSpec((1, 3 * D_H), lambda i: (0, 0)),
        ],
        out_specs=pl.BlockSpec((bm, 3 * D_H), lambda i: (i, 0)),
        out_shape=jax.ShapeDtypeStruct((N_NODES, 3 * D_H), jnp.float32),
    )(nodes, W3, b3)


def _mm_bias_body(x_ref, w_ref, b_ref, o_ref):
    o_ref[...] = (
        jnp.dot(x_ref[...], w_ref[...], preferred_element_type=jnp.float32)
        + b_ref[...]
    )


def _edge_matmul(x, w, b, grid=32):
    e, k = x.shape
    n = w.shape[1]
    bm = e // grid
    return pl.pallas_call(
        _mm_bias_body,
        grid=(grid,),
        in_specs=[
            pl.BlockSpec((bm, k), lambda i: (i, 0)),
            pl.BlockSpec((k, n), lambda i: (0, 0)),
            pl.BlockSpec((1, n), lambda i: (0, 0)),
        ],
        out_specs=pl.BlockSpec((bm, n), lambda i: (i, 0)),
        out_shape=jax.ShapeDtypeStruct((e, n), jnp.float32),
    )(x, w, b)


def _node_update_body(n_ref, ss_ref, sr_ref, cs_ref, cr_ref, wv_ref, bv_ref,
                      wus_ref, wur_ref, n_out, qs_out, qr_out):
    cs = jnp.maximum(cs_ref[:, :1], 1.0)
    cr = jnp.maximum(cr_ref[:, :1], 1.0)
    x = jnp.concatenate([n_ref[...], ss_ref[...] / cs, sr_ref[...] / cr],
                        axis=1)
    h = jnp.maximum(
        jnp.dot(x, wv_ref[...], preferred_element_type=jnp.float32)
        + bv_ref[...], 0.0)
    n_out[...] = h
    qs_out[...] = jnp.dot(h, wus_ref[...], preferred_element_type=jnp.float32)
    qr_out[...] = jnp.dot(h, wur_ref[...], preferred_element_type=jnp.float32)


def _node_update(n, ss, sr, cs, cr, Wv, bv, Wus, Wur):
    grid = 10
    bm = N_NODES // grid
    return pl.pallas_call(
        _node_update_body,
        grid=(grid,),
        in_specs=[
            pl.BlockSpec((bm, D_H), lambda i: (i, 0)),
            pl.BlockSpec((bm, D_H), lambda i: (i, 0)),
            pl.BlockSpec((bm, D_H), lambda i: (i, 0)),
            pl.BlockSpec((bm, D_H), lambda i: (i, 0)),
            pl.BlockSpec((bm, D_H), lambda i: (i, 0)),
            pl.BlockSpec((3 * D_H, D_H), lambda i: (0, 0)),
            pl.BlockSpec((1, D_H), lambda i: (0, 0)),
            pl.BlockSpec((D_H, D_H), lambda i: (0, 0)),
            pl.BlockSpec((D_H, D_H), lambda i: (0, 0)),
        ],
        out_specs=[
            pl.BlockSpec((bm, D_H), lambda i: (i, 0)),
            pl.BlockSpec((bm, D_H), lambda i: (i, 0)),
            pl.BlockSpec((bm, D_H), lambda i: (i, 0)),
        ],
        out_shape=[
            jax.ShapeDtypeStruct((N_NODES, D_H), jnp.float32),
            jax.ShapeDtypeStruct((N_NODES, D_H), jnp.float32),
            jax.ShapeDtypeStruct((N_NODES, D_H), jnp.float32),
        ],
    )(n, ss, sr, cs, cr, Wv, bv, Wus, Wur)


def _sc_mesh():
    return plsc.VectorSubcoreMesh(core_axis_name="c", subcore_axis_name="s",
                                  num_cores=NC, num_subcores=NS)


CPW = (N_CHUNKS + NW - 1) // NW
CPW_LAST = N_CHUNKS - (NW - 1) * CPW


def _gather_pipeline(emm, qs, qr, senders, receivers,
                     idx_s_all, idx_r_all, acc, rows_s, rows_r,
                     sem_e, sem_s, sem_r, process,
                     chunk=CHUNK, n_chunks=N_CHUNKS):
    wid = lax.axis_index("s") * NC + lax.axis_index("c")
    cpw = (n_chunks + NW - 1) // NW
    cpw_last = n_chunks - (NW - 1) * cpw
    first = wid * cpw

    @pl.when(wid < NW - 1)
    def _():
        pltpu.sync_copy(senders.at[pl.ds(first * chunk, cpw * chunk)],
                        idx_s_all)
        pltpu.sync_copy(receivers.at[pl.ds(first * chunk, cpw * chunk)],
                        idx_r_all)

    @pl.when(wid == NW - 1)
    def _():
        pltpu.sync_copy(senders.at[pl.ds(first * chunk, cpw_last * chunk)],
                        idx_s_all.at[pl.ds(0, cpw_last * chunk)])
        pltpu.sync_copy(receivers.at[pl.ds(first * chunk, cpw_last * chunk)],
                        idx_r_all.at[pl.ds(0, cpw_last * chunk)])

    def valid(k):
        return (k < cpw) & (first + k < n_chunks)

    def issue(k, b):
        @pl.when(valid(k))
        def _():
            base = (first + k) * chunk
            pltpu.async_copy(emm.at[pl.ds(base, chunk)], acc[b], sem_e[b])
            pltpu.async_copy(qs.at[idx_s_all.at[pl.ds(k * chunk, chunk)]],
                             rows_s[b], sem_s[b])
            pltpu.async_copy(qr.at[idx_r_all.at[pl.ds(k * chunk, chunk)]],
                             rows_r[b], sem_r[b])

    def finish(k, b):
        @pl.when(valid(k))
        def _():
            base = (first + k) * chunk
            pltpu.make_async_copy(emm.at[pl.ds(base, chunk)], acc[b],
                                  sem_e[b]).wait()
            pltpu.make_async_copy(
                qs.at[idx_s_all.at[pl.ds(k * chunk, chunk)]], rows_s[b],
                sem_s[b]).wait()
            pltpu.make_async_copy(
                qr.at[idx_r_all.at[pl.ds(k * chunk, chunk)]], rows_r[b],
                sem_r[b]).wait()
            process(b, base)

    issue(0, 0)

    def pair_body(m, _):
        k = 2 * m
        issue(k + 1, 1)
        finish(k, 0)
        issue(k + 2, 0)
        finish(k + 1, 1)
        return 0

    lax.fori_loop(0, (cpw + 1) // 2, pair_body, 0)


def _assemble_kernel(emm, qs, qr, senders, receivers, out,
                     idx_s_all, idx_r_all, acc, rows_s, rows_r,
                     sem_e, sem_s, sem_r):

    def process(b, base):
        def row_body(i, _):
            for j in range(D_H // L):
                sl = pl.ds(j * L, L)
                acc[b][i, sl] = jnp.maximum(
                    acc[b][i, sl] + rows_s[b][i, sl] + rows_r[b][i, sl],
                    0.0)
            return 0

        lax.fori_loop(0, CHUNK, row_body, 0)
        pltpu.sync_copy(acc[b], out.at[pl.ds(base, CHUNK)])

    _gather_pipeline(emm, qs, qr, senders, receivers,
                     idx_s_all, idx_r_all, acc, rows_s, rows_r,
                     sem_e, sem_s, sem_r, process)


_ASM_SCRATCH = [
    pltpu.VMEM((CPW * CHUNK,), jnp.int32),
    pltpu.VMEM((CPW * CHUNK,), jnp.int32),
    [pltpu.VMEM((CHUNK, D_H), jnp.float32) for _ in range(2)],
    [pltpu.VMEM((CHUNK, D_H), jnp.float32) for _ in range(2)],
    [pltpu.VMEM((CHUNK, D_H), jnp.float32) for _ in range(2)],
    [pltpu.SemaphoreType.DMA for _ in range(2)],
    [pltpu.SemaphoreType.DMA for _ in range(2)],
    [pltpu.SemaphoreType.DMA for _ in range(2)],
]


def _sc_assemble(emm, qs, qr, senders, receivers):
    return pl.kernel(
        _assemble_kernel,
        out_type=jax.ShapeDtypeStruct((N_EDGES, D_H), jnp.float32),
        mesh=_sc_mesh(),
        scratch_types=_ASM_SCRATCH,
    )(emm, qs, qr, senders, receivers)


def _tile_rows(s, fn):
    for off in (0, 128, 256, 384):
        fn(off, 128)

    @pl.when(s < NS - 1)
    def _():
        fn(512, 112)

    @pl.when(s == NS - 1)
    def _():
        fn(512, 128)


CPT = (N_CHUNKS + NS - 1) // NS


def _scatter_kernel(e, idx_both, s_out, acc_sh, ebuf, idx2, sem_e, sem_i):
    c = lax.axis_index("c")
    s = lax.axis_index("s")

    def zrow(i, _):
        for j in range(D_H // L):
            ebuf[0][i, pl.ds(j * L, L)] = jnp.zeros((L,), jnp.float32)
        return 0

    lax.fori_loop(0, CHUNK, zrow, 0)
    base = s * ROWS_PER_TILE

    def zcopy(off, nrows):
        pltpu.sync_copy(ebuf[0].at[pl.ds(0, nrows)],
                        acc_sh.at[pl.ds(base + off, nrows)])

    _tile_rows(s, zcopy)
    plsc.subcore_barrier()

    first = s * CPT

    def valid(k):
        return (k < CPT) & (first + k < N_CHUNKS)

    def issue(k, b):
        @pl.when(valid(k))
        def _():
            ebase = (first + k) * CHUNK
            pltpu.async_copy(e.at[pl.ds(ebase, CHUNK)], ebuf[b], sem_e[b])
            pltpu.async_copy(idx_both.at[c, pl.ds(ebase, CHUNK)],
                             idx2[b].at[0], sem_i[b])

    def finish(k, b):
        @pl.when(valid(k))
        def _():
            ebase = (first + k) * CHUNK
            pltpu.make_async_copy(e.at[pl.ds(ebase, CHUNK)], ebuf[b],
                                  sem_e[b]).wait()
            pltpu.make_async_copy(idx_both.at[c, pl.ds(ebase, CHUNK)],
                                  idx2[b].at[0], sem_i[b]).wait()
            pltpu.sync_copy(ebuf[b], acc_sh.at[idx2[b].at[0]], add=True)

    issue(0, 0)

    def pair_body(m, _):
        k = 2 * m
        issue(k + 1, 1)
        finish(k, 0)
        issue(k + 2, 0)
        finish(k + 1, 1)
        return 0

    lax.fori_loop(0, (CPT + 1) // 2, pair_body, 0)
    plsc.subcore_barrier()

    def wcopy(off, nrows):
        pltpu.sync_copy(acc_sh.at[pl.ds(base + off, nrows)],
                        ebuf[0].at[pl.ds(0, nrows)])
        pltpu.sync_copy(ebuf[0].at[pl.ds(0, nrows)],
                        s_out.at[c, pl.ds(base + off, nrows)])

    _tile_rows(s, wcopy)


def _sc_scatter(e, idx_both):
    return pl.kernel(
        _scatter_kernel,
        out_type=jax.ShapeDtypeStruct((NC, N_NODES, D_H), jnp.float32),
        mesh=_sc_mesh(),
        scratch_types=[
            pltpu.VMEM_SHARED((N_NODES, D_H), jnp.float32),
            [pltpu.VMEM((CHUNK, D_H), jnp.float32) for _ in range(2)],
            [pltpu.VMEM((1, CHUNK), jnp.int32) for _ in range(2)],
            [pltpu.SemaphoreType.DMA for _ in range(2)],
            [pltpu.SemaphoreType.DMA for _ in range(2)],
        ],
    )(e, idx_both)


def _count_kernel(idx_both, c_out, cnt_sh, cbuf, idx2, sem_i):
    c = lax.axis_index("c")
    s = lax.axis_index("s")

    def zrow(i, _):
        for j in range(D_H // L):
            cbuf[i, pl.ds(j * L, L)] = jnp.zeros((L,), jnp.float32)
        return 0

    lax.fori_loop(0, CHUNK, zrow, 0)
    base = s * ROWS_PER_TILE

    def zcopy(off, nrows):
        pltpu.sync_copy(cbuf.at[pl.ds(0, nrows)],
                        cnt_sh.at[pl.ds(base + off, nrows)])

    _tile_rows(s, zcopy)
    plsc.subcore_barrier()

    def orow(i, _):
        for j in range(D_H // L):
            cbuf[i, pl.ds(j * L, L)] = jnp.ones((L,), jnp.float32)
        return 0

    lax.fori_loop(0, CHUNK, orow, 0)

    first = s * CPT

    def valid(k):
        return (k < CPT) & (first + k < N_CHUNKS)

    def issue(k, b):
        @pl.when(valid(k))
        def _():
            ebase = (first + k) * CHUNK
            pltpu.async_copy(idx_both.at[c, pl.ds(ebase, CHUNK)],
                             idx2[b].at[0], sem_i[b])

    def finish(k, b):
        @pl.when(valid(k))
        def _():
            ebase = (first + k) * CHUNK
            pltpu.make_async_copy(idx_both.at[c, pl.ds(ebase, CHUNK)],
                                  idx2[b].at[0], sem_i[b]).wait()
            pltpu.sync_copy(cbuf, cnt_sh.at[idx2[b].at[0]], add=True)

    issue(0, 0)

    def pair_body(m, _):
        k = 2 * m
        issue(k + 1, 1)
        finish(k, 0)
        issue(k + 2, 0)
        finish(k + 1, 1)
        return 0

    lax.fori_loop(0, (CPT + 1) // 2, pair_body, 0)
    plsc.subcore_barrier()

    def wcopy(off, nrows):
        pltpu.sync_copy(cnt_sh.at[pl.ds(base + off, nrows)],
                        cbuf.at[pl.ds(0, nrows)])
        pltpu.sync_copy(cbuf.at[pl.ds(0, nrows)],
                        c_out.at[c, pl.ds(base + off, nrows)])

    _tile_rows(s, wcopy)


def _sc_count(idx_both):
    return pl.kernel(
        _count_kernel,
        out_type=jax.ShapeDtypeStruct((NC, N_NODES, D_H), jnp.float32),
        mesh=_sc_mesh(),
        scratch_types=[
            pltpu.VMEM_SHARED((N_NODES, D_H), jnp.float32),
            pltpu.VMEM((CHUNK, D_H), jnp.float32),
            [pltpu.VMEM((1, CHUNK), jnp.int32) for _ in range(2)],
            [pltpu.SemaphoreType.DMA for _ in range(2)],
        ],
    )(idx_both)


def kernel(nodes, edges, senders, receivers, Wn, bn, We, be, Wv, bv, Wu, bu,
           Wf, bf):
    senders = senders.astype(jnp.int32)
    receivers = receivers.astype(jnp.int32)

    We_e = We[:D_EDGE]
    We_s = We[D_EDGE:D_EDGE + D_NODE]
    We_r = We[D_EDGE + D_NODE:]
    Wu_e, Wu_s, Wu_r = Wu[:D_H], Wu[D_H:2 * D_H], Wu[2 * D_H:]

    W3 = jnp.concatenate([Wn, We_s, We_r], axis=1)
    b3 = jnp.concatenate([bn, jnp.zeros((2 * D_H,), jnp.float32)])[None, :]

    prep = _node_prep(nodes, W3, b3)
    n = prep[:, :D_H]
    ps, pr = prep[:, D_H:2 * D_H], prep[:, 2 * D_H:]

    idx_both = jnp.stack([senders, receivers])
    c_acc = _sc_count(idx_both)

    e_base = _edge_matmul(edges, We_e, be[None, :])
    e = _sc_assemble(e_base, ps, pr, senders, receivers)

    for _ in range(2):
        s_acc = _sc_scatter(e, idx_both)
        emm = _edge_matmul(e, Wu_e, bu[None, :])
        n, qs, qr = _node_update(n, s_acc[0], s_acc[1], c_acc[0], c_acc[1],
                                 Wv, bv[None, :], Wu_s, Wu_r)
        e = _sc_assemble(emm, qs, qr, senders, receivers)

    vij = _edge_matmul(e, Wf, bf[None, :])
    return vij[:, 0]
